# baseline (speedup 1.0000x reference)
import functools

import jax
import jax.numpy as jnp
from jax.experimental import pallas as pl
from jax.experimental.pallas import tpu as pltpu

_TB = 4096


def _body(ca_ref, cb_ref, w_in_ref, b_in_ref, w1_ref, b1_ref, w2_ref, b2_ref,
          w_out_ref, b_out_ref, o_ref):
    w_in = w_in_ref[...]
    b_in = b_in_ref[...]
    x0 = jnp.maximum(
        jnp.dot(ca_ref[...], w_in, preferred_element_type=jnp.float32) + b_in,
        0.0)
    x1 = jnp.maximum(
        jnp.dot(cb_ref[...], w_in, preferred_element_type=jnp.float32) + b_in,
        0.0)
    xm = x0 + x1
    y = jnp.maximum(
        jnp.dot(xm, 0.5 * w1_ref[...], preferred_element_type=jnp.float32)
        + b1_ref[...], 0.0)
    z = jnp.maximum(
        jnp.dot(y, w2_ref[...], preferred_element_type=jnp.float32)
        + b2_ref[...], 0.0)
    w_eff = w_out_ref[:64, :] + w_out_ref[64:, :]
    s = jnp.dot(z, w_eff, preferred_element_type=jnp.float32) + b_out_ref[...]
    o_ref[...] = jax.nn.sigmoid(s)


def kernel(combined, fc_in_w, fc_in_b, conv1_w, conv1_b, conv2_w, conv2_b,
           fc_out_w, fc_out_b):
    B = combined.shape[0]
    d = fc_in_w.shape[0]
    grid = (B // _TB,)
    full = lambda shape: pl.BlockSpec(shape, lambda i: (0, 0))
    out = pl.pallas_call(
        _body,
        grid=grid,
        in_specs=[
            pl.BlockSpec((_TB, d), lambda i: (i, 0)),
            pl.BlockSpec((_TB, d), lambda i: (i, 1)),
            full(fc_in_w.shape),
            full((1, 128)),
            full(conv1_w.shape),
            full((1, 128)),
            full(conv2_w.shape),
            full((1, 64)),
            full(fc_out_w.shape),
            full((1, 1)),
        ],
        out_specs=pl.BlockSpec((_TB, 1), lambda i: (i, 0)),
        out_shape=jax.ShapeDtypeStruct((B, 1), jnp.float32),
        compiler_params=pltpu.CompilerParams(
            dimension_semantics=("parallel",)),
    )(combined, combined, fc_in_w, fc_in_b.reshape(1, 128), conv1_w,
      conv1_b.reshape(1, 128), conv2_w, conv2_b.reshape(1, 64), fc_out_w,
      fc_out_b.reshape(1, 1))
    return out


# speedup vs baseline: 1.0047x; 1.0047x over previous
import functools

import jax
import jax.numpy as jnp
from jax.experimental import pallas as pl
from jax.experimental.pallas import tpu as pltpu

_TB = 4096


def _body(c_ref, w_in_ref, b_in_ref, w1_ref, b1_ref, w2_ref, b2_ref,
          w_out_ref, b_out_ref, o_ref, *, d):
    c = c_ref[...]
    w_in = w_in_ref[...]
    b_in = b_in_ref[...]
    x0 = jnp.maximum(
        jnp.dot(c[:, :d], w_in, preferred_element_type=jnp.float32) + b_in, 0.0)
    x1 = jnp.maximum(
        jnp.dot(c[:, d:], w_in, preferred_element_type=jnp.float32) + b_in, 0.0)
    xm = x0 + x1
    y = jnp.maximum(
        jnp.dot(xm, 0.5 * w1_ref[...], preferred_element_type=jnp.float32)
        + b1_ref[...], 0.0)
    z = jnp.maximum(
        jnp.dot(y, w2_ref[...], preferred_element_type=jnp.float32)
        + b2_ref[...], 0.0)
    w_eff = w_out_ref[:64, :] + w_out_ref[64:, :]
    s = jnp.dot(z, w_eff, preferred_element_type=jnp.float32) + b_out_ref[...]
    o_ref[...] = jax.nn.sigmoid(s)


def kernel(combined, fc_in_w, fc_in_b, conv1_w, conv1_b, conv2_w, conv2_b,
           fc_out_w, fc_out_b):
    B = combined.shape[0]
    d = fc_in_w.shape[0]
    grid = (B // _TB,)
    full = lambda shape: pl.BlockSpec(shape, lambda i: (0, 0))
    out = pl.pallas_call(
        functools.partial(_body, d=d),
        grid=grid,
        in_specs=[
            pl.BlockSpec((_TB, 2 * d), lambda i: (i, 0)),
            full(fc_in_w.shape),
            full((1, 128)),
            full(conv1_w.shape),
            full((1, 128)),
            full(conv2_w.shape),
            full((1, 64)),
            full(fc_out_w.shape),
            full((1, 1)),
        ],
        out_specs=pl.BlockSpec((_TB, 1), lambda i: (i, 0)),
        out_shape=jax.ShapeDtypeStruct((B, 1), jnp.float32),
        compiler_params=pltpu.CompilerParams(
            dimension_semantics=("parallel",)),
    )(combined, fc_in_w, fc_in_b.reshape(1, 128), conv1_w,
      conv1_b.reshape(1, 128), conv2_w, conv2_b.reshape(1, 64), fc_out_w,
      fc_out_b.reshape(1, 1))
    return out
